# speedup vs baseline: 15.5634x; 15.5634x over previous
import functools

import jax
import jax.numpy as jnp
from jax import lax
from jax.experimental import pallas as pl
from jax.experimental.pallas import tpu as pltpu
from jax.experimental.pallas import tpu_sc as plsc

N = 10000
E = 320000
D = 128

NC, NS = 2, 16
NW = NC * NS
CH = 128
CPT = -(-E // (CH * NW))
E_PAD = CPT * CH * NW
NP = 10240
RPT = NP // NS

_MESH = plsc.VectorSubcoreMesh(
    core_axis_name="c", subcore_axis_name="s", num_cores=NC, num_subcores=NS
)


def _worker():
    return lax.axis_index("s") * NC + lax.axis_index("c")


def _deg_body(col_hbm, deg0_hbm, deg1_hbm, ones_v, idx_v, zrow_v, deg_sh, sem):
    c = lax.axis_index("c")
    s = lax.axis_index("s")
    w = _worker()

    def fill(i, _):
        ones_v[pl.ds(i * 16, 16)] = jnp.ones((16,), jnp.float32)
        zrow_v[pl.ds(i * 16, 16)] = jnp.zeros((16,), jnp.float32)
        return 0

    lax.fori_loop(0, CH // 16, fill, 0)

    def zloop(i, _):
        pltpu.sync_copy(zrow_v, deg_sh.at[pl.ds(s * RPT + i * CH, CH)])
        return 0

    lax.fori_loop(0, RPT // CH, zloop, 0)
    plsc.subcore_barrier()

    def body(j, _):
        base = pl.multiple_of((w * CPT + j) * CH, CH)
        pltpu.sync_copy(col_hbm.at[pl.ds(base, CH)], idx_v)
        pltpu.sync_copy(ones_v, deg_sh.at[idx_v], add=True)
        return 0

    lax.fori_loop(0, CPT, body, 0)
    plsc.subcore_barrier()

    @pl.when(c == 0)
    def _():
        pltpu.sync_copy(deg_sh.at[pl.ds(s * RPT, RPT)],
                        deg0_hbm.at[pl.ds(s * RPT, RPT)])

    @pl.when(c == 1)
    def _():
        pltpu.sync_copy(deg_sh.at[pl.ds(s * RPT, RPT)],
                        deg1_hbm.at[pl.ds(s * RPT, RPT)])


_deg_call = functools.partial(
    pl.kernel,
    out_type=(
        jax.ShapeDtypeStruct((NP,), jnp.float32),
        jax.ShapeDtypeStruct((NP,), jnp.float32),
    ),
    mesh=_MESH,
    scratch_types=[
        pltpu.VMEM((CH,), jnp.float32),
        pltpu.VMEM((CH,), jnp.int32),
        pltpu.VMEM((CH,), jnp.float32),
        pltpu.VMEM_SHARED((NP,), jnp.float32),
        pltpu.SemaphoreType.DMA,
    ],
)(_deg_body)


def _mm_body(x_ref, w_ref, d0_ref, d1_ref, h2_ref, dinv_ref):
    deg = d0_ref[...] + d1_ref[...] + 1.0
    dinv = lax.rsqrt(deg)
    h = jnp.dot(x_ref[...], w_ref[...], preferred_element_type=jnp.float32)
    h2_ref[...] = h * dinv
    dinv_ref[...] = dinv


_MMR = 2000


def _mm_call(x, W, d0, d1):
    grid = N // _MMR
    return pl.pallas_call(
        _mm_body,
        grid=(grid,),
        in_specs=[
            pl.BlockSpec((_MMR, D), lambda i: (i, 0)),
            pl.BlockSpec((D, D), lambda i: (0, 0)),
            pl.BlockSpec((_MMR, 1), lambda i: (i, 0)),
            pl.BlockSpec((_MMR, 1), lambda i: (i, 0)),
        ],
        out_specs=[
            pl.BlockSpec((_MMR, D), lambda i: (i, 0)),
            pl.BlockSpec((_MMR, 1), lambda i: (i, 0)),
        ],
        out_shape=[
            jax.ShapeDtypeStruct((N, D), jnp.float32),
            jax.ShapeDtypeStruct((N, 1), jnp.float32),
        ],
    )(x, W, d0, d1)


def _scat_body(row_hbm, col_hbm, h2_hbm, p0_hbm, p1_hbm,
               ridx_v, cidx_v, rows_v, zbuf_v, acc_sh, sem):
    c = lax.axis_index("c")
    s = lax.axis_index("s")
    w = _worker()

    def zb(i, _):
        def zb2(j, _):
            zbuf_v[i, pl.ds(j * 16, 16)] = jnp.zeros((16,), jnp.float32)
            return 0
        lax.fori_loop(0, D // 16, zb2, 0)
        return 0

    lax.fori_loop(0, CH, zb, 0)

    def zloop(k, _):
        pltpu.sync_copy(zbuf_v, acc_sh.at[pl.ds(s * RPT + k * CH, CH)])
        return 0

    lax.fori_loop(0, RPT // CH, zloop, 0)
    plsc.subcore_barrier()

    def body(j, _):
        base = pl.multiple_of((w * CPT + j) * CH, CH)
        pltpu.sync_copy(row_hbm.at[pl.ds(base, CH)], ridx_v)
        pltpu.sync_copy(col_hbm.at[pl.ds(base, CH)], cidx_v)
        pltpu.async_copy(h2_hbm.at[ridx_v], rows_v, sem).wait()
        pltpu.sync_copy(rows_v, acc_sh.at[cidx_v], add=True)
        return 0

    lax.fori_loop(0, CPT, body, 0)
    plsc.subcore_barrier()

    @pl.when(c == 0)
    def _():
        pltpu.sync_copy(acc_sh.at[pl.ds(s * RPT, RPT)],
                        p0_hbm.at[pl.ds(s * RPT, RPT)])

    @pl.when(c == 1)
    def _():
        pltpu.sync_copy(acc_sh.at[pl.ds(s * RPT, RPT)],
                        p1_hbm.at[pl.ds(s * RPT, RPT)])


_scat_call = functools.partial(
    pl.kernel,
    out_type=(
        jax.ShapeDtypeStruct((NP, D), jnp.float32),
        jax.ShapeDtypeStruct((NP, D), jnp.float32),
    ),
    mesh=_MESH,
    scratch_types=[
        pltpu.VMEM((CH,), jnp.int32),
        pltpu.VMEM((CH,), jnp.int32),
        pltpu.VMEM((CH, D), jnp.float32),
        pltpu.VMEM((CH, D), jnp.float32),
        pltpu.VMEM_SHARED((NP, D), jnp.float32),
        pltpu.SemaphoreType.DMA,
    ],
)(_scat_body)


def _comb_body(p0_ref, p1_ref, h2_ref, dinv_ref, b_ref, out_ref):
    out_ref[...] = (
        dinv_ref[...] * (p0_ref[...] + p1_ref[...] + h2_ref[...]) + b_ref[...]
    )


def _comb_call(p0, p1, h2, dinv, b2):
    grid = N // _MMR
    return pl.pallas_call(
        _comb_body,
        grid=(grid,),
        in_specs=[
            pl.BlockSpec((_MMR, D), lambda i: (i, 0)),
            pl.BlockSpec((_MMR, D), lambda i: (i, 0)),
            pl.BlockSpec((_MMR, D), lambda i: (i, 0)),
            pl.BlockSpec((_MMR, 1), lambda i: (i, 0)),
            pl.BlockSpec((1, D), lambda i: (0, 0)),
        ],
        out_specs=pl.BlockSpec((_MMR, D), lambda i: (i, 0)),
        out_shape=jax.ShapeDtypeStruct((N, D), jnp.float32),
    )(p0, p1, h2, dinv, b2)


def kernel(x, edge_index, W, b):
    row = edge_index[0]
    col = edge_index[1]
    pad = E_PAD - E
    row_p = jnp.concatenate([row, jnp.zeros((pad,), jnp.int32)])
    col_p = jnp.concatenate([col, jnp.full((pad,), N, jnp.int32)])

    deg0, deg1 = _deg_call(col_p)
    d0 = deg0[:N, None]
    d1 = deg1[:N, None]
    h2, dinv = _mm_call(x, W, d0, d1)
    p0, p1 = _scat_call(row_p, col_p, h2)
    b2 = b[None, :]
    return _comb_call(p0, p1, h2, dinv, b2)
